# baseline (speedup 1.0000x reference)
import functools

import jax
import jax.numpy as jnp
from jax import lax
from jax.experimental import pallas as pl
from jax.experimental.pallas import tpu as pltpu
from jax.experimental.pallas import tpu_sc as plsc

N_NODES = 10000
N_EDGES = 160000
D = 256
H = 128
NC = 2
NS = 16
NPASS = 2
PR = 5120
AR = PR + 8 * NS
EPT = N_EDGES // NS
CH = 80
NCH = EPT // CH
RPT = PR // NS
ZROWS = 160


def _agg_body(x2_hbm, src2_hbm, dstp_hbm, out_hbm,
              src_idx_v, dst_idx_v, rows_v, zbuf_v, agg_sh, sem):
    c = lax.axis_index("c")
    s = lax.axis_index("s")

    def zrow(i, _):
        for j in range(H // 16):
            zbuf_v[i, pl.ds(16 * j, 16)] = jnp.zeros((16,), jnp.float32)
        return _
    lax.fori_loop(0, ZROWS, zrow, None)

    for p in range(NPASS):
        for z in range(RPT // ZROWS):
            pltpu.sync_copy(zbuf_v,
                            agg_sh.at[pl.ds(s * RPT + z * ZROWS, ZROWS)])
        plsc.subcore_barrier()

        pltpu.sync_copy(src2_hbm.at[p, c, s], src_idx_v)
        pltpu.sync_copy(dstp_hbm.at[p, s], dst_idx_v)

        def chunk(j, _):
            pltpu.async_copy(x2_hbm.at[src_idx_v.at[j]], rows_v, sem).wait()
            pltpu.sync_copy(rows_v, agg_sh.at[dst_idx_v.at[j]], add=True)
            return _
        lax.fori_loop(0, NCH, chunk, None)

        plsc.subcore_barrier()
        pltpu.sync_copy(agg_sh.at[pl.ds(s * RPT, RPT)],
                        out_hbm.at[c, p, pl.ds(s * RPT, RPT)])


@functools.cache
def _agg():
    return pl.kernel(
        _agg_body,
        out_type=jax.ShapeDtypeStruct((NC, NPASS, PR, H), jnp.float32),
        mesh=plsc.VectorSubcoreMesh(core_axis_name="c", subcore_axis_name="s",
                                    num_cores=NC, num_subcores=NS),
        scratch_types=[
            pltpu.VMEM((NCH, CH), jnp.int32),
            pltpu.VMEM((NCH, CH), jnp.int32),
            pltpu.VMEM((CH, H), jnp.float32),
            pltpu.VMEM((ZROWS, H), jnp.float32),
            pltpu.VMEM_SHARED((AR, H), jnp.float32),
            pltpu.SemaphoreType.DMA,
        ],
    )


BM = 1000


def _mlp_body(x_ref, al_ref, ar_ref, w1_ref, b1_ref, w2_ref, b2_ref, o_ref):
    h = x_ref[...] + jnp.concatenate([al_ref[0], ar_ref[0]], axis=-1)
    t = jnp.dot(h, w1_ref[...], preferred_element_type=jnp.float32)
    t = jnp.maximum(t + b1_ref[...], 0.0)
    o = jnp.dot(t, w2_ref[...], preferred_element_type=jnp.float32)
    o_ref[...] = o + b2_ref[...]


def _mlp(x, agg2, W1, b1, W2, b2):
    return pl.pallas_call(
        _mlp_body,
        grid=(N_NODES // BM,),
        in_specs=[
            pl.BlockSpec((BM, D), lambda i: (i, 0)),
            pl.BlockSpec((1, BM, H), lambda i: (0, i, 0)),
            pl.BlockSpec((1, BM, H), lambda i: (1, i, 0)),
            pl.BlockSpec((D, D), lambda i: (0, 0)),
            pl.BlockSpec((1, D), lambda i: (0, 0)),
            pl.BlockSpec((D, D), lambda i: (0, 0)),
            pl.BlockSpec((1, D), lambda i: (0, 0)),
        ],
        out_specs=pl.BlockSpec((BM, D), lambda i: (i, 0)),
        out_shape=jax.ShapeDtypeStruct((N_NODES, D), jnp.float32),
    )(x, agg2, agg2, W1, b1, W2, b2)


def kernel(x, edge_index, W1, b1, W2, b2):
    src = edge_index[0].astype(jnp.int32)
    dst = edge_index[1].astype(jnp.int32)
    x2 = x.reshape(2 * N_NODES, H)
    tile_trash = PR + 8 * (jnp.arange(N_EDGES, dtype=jnp.int32) // EPT)
    srcs, dsts = [], []
    for p in range(NPASS):
        in_range = (dst >= p * PR) & (dst < (p + 1) * PR)
        dsts.append(jnp.where(in_range, dst - p * PR, tile_trash))
        sp = jnp.where(in_range, 2 * src, 0)
        srcs.append(jnp.stack([sp, sp + 1]))
    src2 = jnp.stack(srcs).reshape(NPASS, NC, NS, NCH, CH)
    dstp = jnp.stack(dsts).reshape(NPASS, NS, NCH, CH)
    agg2 = _agg()(x2, src2, dstp).reshape(NC, NPASS * PR, H)
    return _mlp(x, agg2, W1, b1.reshape(1, D), W2, b2.reshape(1, D))
